# Initial kernel scaffold; baseline (speedup 1.0000x reference)
import jax
import jax.numpy as jnp
from jax.experimental import pallas as pl


def kernel(id_sparse, id_seq, W_user, W_hist):
    raise NotImplementedError("write your pallas kernel here")


# speedup vs baseline: 1.5239x; 1.5239x over previous
import functools

import jax
import jax.numpy as jnp
from jax import lax
from jax.experimental import pallas as pl
from jax.experimental.pallas import tpu as pltpu
from jax.experimental.pallas import tpu_sc as plsc

_B, _L, _V, _D = 16384, 50, 1000000, 32
_NC, _NS = 2, 16
_NW = _NC * _NS
_BPW = _B // _NW
_CH = 16
_NCHUNK = _BPW // _CH
_G = 80
_NG = (_CH * _L) // _G


def _build_sc_call():
  mesh = plsc.VectorSubcoreMesh(core_axis_name="c", subcore_axis_name="s")

  @functools.partial(
      pl.kernel,
      out_type=jax.ShapeDtypeStruct((_B * 2 * _D,), jnp.float32),
      mesh=mesh,
      compiler_params=pltpu.CompilerParams(
          needs_layout_passes=False, use_tc_tiling_on_sc=False),
      scratch_types=[
          pltpu.VMEM((_CH * _L,), jnp.int32),
          pltpu.VMEM((_CH,), jnp.int32),
          pltpu.VMEM((_CH * _L, _D), jnp.float32),
          pltpu.VMEM((_CH, _D), jnp.float32),
          pltpu.VMEM((_CH * 2 * _D,), jnp.float32),
          pltpu.VMEM((16, _D), jnp.float32),
          pltpu.VMEM((_CH,), jnp.float32),
          pltpu.VMEM((_CH,), jnp.float32),
          pltpu.SemaphoreType.DMA,
      ],
  )
  def sc_kernel(ids_hbm, seq_hbm, wu_hbm, wh_hbm, out_hbm,
                seq_v, idu_v, rows_v, urows_v, out_v, w0_v, n0_v, inv_v, sem):
    wid = lax.axis_index("s") * _NC + lax.axis_index("c")
    base_w = wid * _BPW
    lanes = lax.iota(jnp.int32, 16)

    pltpu.async_copy(wh_hbm.at[jnp.zeros((16,), jnp.int32)], w0_v, sem).wait()
    w00 = w0_v[0, pl.ds(0, 16)]
    w01 = w0_v[0, pl.ds(16, 16)]

    def chunk_body(ci):
      b0 = base_w + ci * _CH
      pltpu.sync_copy(seq_hbm.at[pl.ds(b0 * _L, _CH * _L)], seq_v)
      pltpu.sync_copy(ids_hbm.at[pl.ds(b0, _CH)], idu_v)
      cps = []
      for g in range(_NG):
        cps.append(pltpu.async_copy(
            wh_hbm.at[seq_v.at[pl.ds(g * _G, _G)]],
            rows_v.at[pl.ds(g * _G, _G)], sem))
      cps.append(pltpu.async_copy(wu_hbm.at[idu_v], urows_v, sem))

      cnt = jnp.zeros((16,), jnp.float32)
      for l in range(_L):
        col = plsc.load_gather(seq_v, [lanes * _L + l])
        cnt = cnt + jnp.where(col == 0, 1.0, 0.0).astype(jnp.float32)
      n0_v[...] = cnt
      inv_v[...] = 1.0 / ((_L - cnt) + 1e-8)

      for cp in cps:
        cp.wait()

      for b in range(_CH):
        bsplat = jnp.full((16,), b, jnp.int32)
        n0 = plsc.load_gather(n0_v, [bsplat])
        inv = plsc.load_gather(inv_v, [bsplat])
        acc0 = jnp.zeros((16,), jnp.float32)
        acc1 = jnp.zeros((16,), jnp.float32)
        for l in range(_L):
          acc0 = acc0 + rows_v[b * _L + l, pl.ds(0, 16)]
          acc1 = acc1 + rows_v[b * _L + l, pl.ds(16, 16)]
        o = b * 2 * _D
        out_v[pl.ds(o, 16)] = urows_v[b, pl.ds(0, 16)]
        out_v[pl.ds(o + 16, 16)] = urows_v[b, pl.ds(16, 16)]
        out_v[pl.ds(o + 32, 16)] = (acc0 - n0 * w00) * inv
        out_v[pl.ds(o + 48, 16)] = (acc1 - n0 * w01) * inv

      pltpu.sync_copy(out_v, out_hbm.at[pl.ds(b0 * 2 * _D, _CH * 2 * _D)])

    pl.loop(0, _NCHUNK)(chunk_body)

  return sc_kernel


_sc_call = _build_sc_call()


def kernel(id_sparse, id_seq, W_user, W_hist):
  out = _sc_call(id_sparse, id_seq.reshape(-1), W_user, W_hist)
  return out.reshape(_B, 2, _D)
